# baseline (speedup 1.0000x reference)
import jax
import jax.numpy as jnp
from jax.experimental import pallas as pl
from jax.experimental.pallas import tpu as pltpu
from jax.sharding import Mesh, PartitionSpec as P

M = 16384
K = 4096
N = 4096

BM = 256


def _mm_kernel(x_ref, w_ref, o_ref):
    o_ref[...] = jax.lax.dot_general(
        x_ref[...].astype(jnp.bfloat16),
        w_ref[...],
        dimension_numbers=(((1,), (1,)), ((), ())),
        preferred_element_type=jnp.float32,
    )


def _pallas_mm(x, w16):
    m_local = x.shape[0]
    bm = min(BM, m_local)
    return pl.pallas_call(
        _mm_kernel,
        grid=(m_local // bm,),
        in_specs=[
            pl.BlockSpec((bm, K), lambda i: (i, 0)),
            pl.BlockSpec((N, K), lambda i: (0, 0)),
        ],
        out_specs=pl.BlockSpec((bm, N), lambda i: (i, 0)),
        out_shape=jax.ShapeDtypeStruct((m_local, N), jnp.float32),
        compiler_params=pltpu.CompilerParams(
            vmem_limit_bytes=128 * 1024 * 1024,
        ),
    )(x, w16)


def _local_mm(x_local, w_local):
    w16 = jax.lax.all_gather(
        w_local.astype(jnp.bfloat16), "d", axis=0, tiled=True
    )
    return _pallas_mm(x_local, w16)


def kernel(x, W):
    devs = jax.devices()
    n = min(len(devs), 8)
    while n > 1 and (M % n or N % n):
        n -= 1
    if n == 1:
        return _pallas_mm(x, W.astype(jnp.bfloat16))
    mesh = Mesh(devs[:n], ("d",))
    f = jax.shard_map(
        _local_mm,
        mesh=mesh,
        in_specs=(P("d", None), P("d", None)),
        out_specs=P("d", None),
        check_vma=False,
    )
    return f(x, W)


# speedup vs baseline: 1.6029x; 1.6029x over previous
import jax
import jax.numpy as jnp
from jax.experimental import pallas as pl
from jax.experimental.pallas import tpu as pltpu

M = 16384
K = 4096
N = 4096

BM = 256
WCHUNK = 128


def _mm_kernel(x_ref, w_hbm_ref, o_ref, w16_ref, bounce_ref, sems_ref):
    @pl.when(pl.program_id(0) == 0)
    def _load_w():
        nc = N // WCHUNK

        def _copy(c, slot):
            return pltpu.make_async_copy(
                w_hbm_ref.at[pl.ds(c * WCHUNK, WCHUNK), :],
                bounce_ref.at[slot],
                sems_ref.at[slot],
            )

        _copy(0, 0).start()
        _copy(1, 1).start()

        def _step(c, slot):
            _copy(c, slot).wait()

            @pl.when(c + 2 < nc)
            def _():
                _copy(c + 2, slot).start()

            w16_ref[pl.ds(c * WCHUNK, WCHUNK), :] = bounce_ref[
                slot
            ].astype(jnp.bfloat16)

        def _body(i, carry):
            _step(2 * i, 0)
            _step(2 * i + 1, 1)
            return carry

        jax.lax.fori_loop(0, nc // 2, _body, 0)

    o_ref[...] = jax.lax.dot_general(
        x_ref[...].astype(jnp.bfloat16),
        w16_ref[...],
        dimension_numbers=(((1,), (1,)), ((), ())),
        preferred_element_type=jnp.float32,
    )


def kernel(x, W):
    return pl.pallas_call(
        _mm_kernel,
        grid=(M // BM,),
        in_specs=[
            pl.BlockSpec((BM, K), lambda i: (i, 0)),
            pl.BlockSpec(memory_space=pl.ANY),
        ],
        out_specs=pl.BlockSpec((BM, N), lambda i: (i, 0)),
        out_shape=jax.ShapeDtypeStruct((M, N), jnp.float32),
        scratch_shapes=[
            pltpu.VMEM((N, K), jnp.bfloat16),
            pltpu.VMEM((2, WCHUNK, K), jnp.float32),
            pltpu.SemaphoreType.DMA((2,)),
        ],
        compiler_params=pltpu.CompilerParams(
            vmem_limit_bytes=128 * 1024 * 1024,
        ),
    )(x, W)


# speedup vs baseline: 1.6215x; 1.0116x over previous
import jax
import jax.numpy as jnp
from jax.experimental import pallas as pl
from jax.experimental.pallas import tpu as pltpu

M = 16384
K = 4096
N = 4096

BM = 256
WCHUNK = 256


def _mm_kernel(x_ref, w_hbm_ref, o_ref, w16_ref, bounce_ref, sems_ref):
    @pl.when(pl.program_id(0) == 0)
    def _load_w():
        nc = N // WCHUNK

        def _copy(c, slot):
            return pltpu.make_async_copy(
                w_hbm_ref.at[pl.ds(c * WCHUNK, WCHUNK), :],
                bounce_ref.at[slot],
                sems_ref.at[slot],
            )

        _copy(0, 0).start()
        _copy(1, 1).start()

        def _step(c, slot):
            _copy(c, slot).wait()

            @pl.when(c + 2 < nc)
            def _():
                _copy(c + 2, slot).start()

            w16_ref[pl.ds(c * WCHUNK, WCHUNK), :] = bounce_ref[
                slot
            ].astype(jnp.bfloat16)

        def _body(i, carry):
            _step(2 * i, 0)
            _step(2 * i + 1, 1)
            return carry

        jax.lax.fori_loop(0, nc // 2, _body, 0)

    o_ref[...] = jax.lax.dot_general(
        x_ref[...].astype(jnp.bfloat16),
        w16_ref[...],
        dimension_numbers=(((1,), (1,)), ((), ())),
        preferred_element_type=jnp.float32,
    )


def kernel(x, W):
    return pl.pallas_call(
        _mm_kernel,
        grid=(M // BM,),
        in_specs=[
            pl.BlockSpec((BM, K), lambda i: (i, 0)),
            pl.BlockSpec(memory_space=pl.ANY),
        ],
        out_specs=pl.BlockSpec((BM, N), lambda i: (i, 0)),
        out_shape=jax.ShapeDtypeStruct((M, N), jnp.float32),
        scratch_shapes=[
            pltpu.VMEM((N, K), jnp.bfloat16),
            pltpu.VMEM((2, WCHUNK, K), jnp.float32),
            pltpu.SemaphoreType.DMA((2,)),
        ],
        compiler_params=pltpu.CompilerParams(
            vmem_limit_bytes=128 * 1024 * 1024,
        ),
    )(x, W)
